# baseline (speedup 1.0000x reference)
import jax
import jax.numpy as jnp
from jax import lax
from jax.experimental import pallas as pl
from jax.experimental.pallas import tpu as pltpu
from jax.experimental.pallas import tpu_sc as plsc

_EMBED = 128
_BATCH = 16384
_NC, _NS, _L = 2, 16, 16
_NW = _NC * _NS
_BPW = _BATCH // _NW
_CH = 128
_NCH = _BPW // _CH


def _log_sigmoid(x):
    u = jnp.exp(-jnp.abs(x))
    z = u / (u + 2.0)
    z2 = z * z
    p = jnp.float32(1.0 / 11.0)
    p = p * z2 + jnp.float32(1.0 / 9.0)
    p = p * z2 + jnp.float32(1.0 / 7.0)
    p = p * z2 + jnp.float32(1.0 / 5.0)
    p = p * z2 + jnp.float32(1.0 / 3.0)
    p = p * z2 + jnp.float32(1.0)
    log1p_u = 2.0 * z * p
    return jnp.minimum(x, 0.0) - log1p_u


_NBUF = 3


def _body(tw_hbm, cw_hbm, wt_hbm, ct_hbm, out_hbm,
          tw_idx, cw_idx, tbuf, cbuf, obuf, sbuf,
          st0, sc0, st1, sc1, st2, sc2):
    cid = lax.axis_index("c")
    sid = lax.axis_index("s")
    wid = sid * _NC + cid

    pltpu.sync_copy(tw_hbm.at[pl.ds(wid * _NCH, _NCH)], tw_idx)
    pltpu.sync_copy(cw_hbm.at[pl.ds(wid * _NCH, _NCH)], cw_idx)

    sems = ((st0, sc0), (st1, sc1), (st2, sc2))

    def fire(j, par):
        st, sc = sems[par]
        ht = pltpu.async_copy(wt_hbm.at[tw_idx.at[j]], tbuf.at[par], st)
        hc = pltpu.async_copy(ct_hbm.at[cw_idx.at[j]], cbuf.at[par], sc)
        return ht, hc

    lanes = lax.iota(jnp.int32, _L)

    def compute(j, par):
        tb = tbuf.at[par]
        cb = cbuf.at[par]

        def abody(i, carry):
            for u in range(2):
                p = i * 2 + u
                acc = tb[p, pl.ds(0, _L)] * cb[p, pl.ds(0, _L)]
                for e in range(1, _EMBED // _L):
                    acc = acc + (tb[p, pl.ds(e * _L, _L)] *
                                 cb[p, pl.ds(e * _L, _L)])
                sbuf[p, pl.ds(0, _L)] = acc
            return carry

        lax.fori_loop(0, _CH // 2, abody, 0)

        def bbody(g, carry):
            rows = lanes + g * _L
            tot = plsc.load_gather(sbuf, [rows, jnp.full((_L,), 0,
                                                         jnp.int32)])
            for c in range(1, _L):
                tot = tot + plsc.load_gather(
                    sbuf, [rows, jnp.full((_L,), c, jnp.int32)])
            obuf[pl.ds(j * _CH + g * _L, _L)] = _log_sigmoid(tot)
            return carry

        lax.fori_loop(0, _CH // _L, bbody, 0)

    handles = [None] * _NCH
    for j in range(_NBUF - 1):
        handles[j] = fire(j, j % _NBUF)
    for j in range(_NCH):
        nxt = j + _NBUF - 1
        if nxt < _NCH:
            handles[nxt] = fire(nxt, nxt % _NBUF)
        ht, hc = handles[j]
        ht.wait()
        hc.wait()
        compute(j, j % _NBUF)

    pltpu.sync_copy(obuf, out_hbm.at[pl.ds(wid * _BPW, _BPW)])


def kernel(target_word, context_word, word_table, context_table):
    tw = target_word.reshape(_NW * _NCH, _CH)
    cw = context_word.reshape(_NW * _NCH, _CH)
    mesh = plsc.VectorSubcoreMesh(core_axis_name="c", subcore_axis_name="s",
                                  num_cores=_NC, num_subcores=_NS)
    return pl.kernel(
        _body,
        out_type=jax.ShapeDtypeStruct((_BATCH,), jnp.float32),
        mesh=mesh,
        compiler_params=pltpu.CompilerParams(needs_layout_passes=False),
        scratch_types=[
            pltpu.VMEM((_NCH, _CH), jnp.int32),
            pltpu.VMEM((_NCH, _CH), jnp.int32),
            pltpu.VMEM((_NBUF, _CH, _EMBED), jnp.float32),
            pltpu.VMEM((_NBUF, _CH, _EMBED), jnp.float32),
            pltpu.VMEM((_BPW,), jnp.float32),
            pltpu.VMEM((_CH, _L + 1), jnp.float32),
            pltpu.SemaphoreType.DMA,
            pltpu.SemaphoreType.DMA,
            pltpu.SemaphoreType.DMA,
            pltpu.SemaphoreType.DMA,
            pltpu.SemaphoreType.DMA,
            pltpu.SemaphoreType.DMA,
        ],
    )(tw, cw, word_table, context_table)


# speedup vs baseline: 1.0382x; 1.0382x over previous
import jax
import jax.numpy as jnp
from jax import lax
from jax.experimental import pallas as pl
from jax.experimental.pallas import tpu as pltpu
from jax.experimental.pallas import tpu_sc as plsc

_EMBED = 128
_BATCH = 16384
_NC, _NS, _L = 2, 16, 16
_NW = _NC * _NS
_BPW = _BATCH // _NW
_CH = 128
_NCH = _BPW // _CH


def _log_sigmoid(x):
    u = jnp.exp(-jnp.abs(x))
    z = u / (u + 2.0)
    z2 = z * z
    p = jnp.float32(1.0 / 11.0)
    p = p * z2 + jnp.float32(1.0 / 9.0)
    p = p * z2 + jnp.float32(1.0 / 7.0)
    p = p * z2 + jnp.float32(1.0 / 5.0)
    p = p * z2 + jnp.float32(1.0 / 3.0)
    p = p * z2 + jnp.float32(1.0)
    log1p_u = 2.0 * z * p
    return jnp.minimum(x, 0.0) - log1p_u


def _body(tw_hbm, cw_hbm, wt_hbm, ct_hbm, out_hbm,
          tw_idx, cw_idx, tbuf, cbuf, obuf, sbuf, st0, sc0, st1, sc1):
    cid = lax.axis_index("c")
    sid = lax.axis_index("s")
    wid = sid * _NC + cid

    hti = pltpu.async_copy(tw_hbm.at[pl.ds(wid * _NCH, _NCH)], tw_idx, st1)
    hci = pltpu.async_copy(cw_hbm.at[pl.ds(wid * _NCH, _NCH)], cw_idx, sc1)

    sems = ((st0, sc0), (st1, sc1))

    def fire(j, par):
        st, sc = sems[par]
        ht = pltpu.async_copy(wt_hbm.at[tw_idx.at[j]], tbuf.at[par], st)
        hc = pltpu.async_copy(ct_hbm.at[cw_idx.at[j]], cbuf.at[par], sc)
        return ht, hc

    lanes = lax.iota(jnp.int32, _L)

    def compute(j, par):
        tb = tbuf.at[par]
        cb = cbuf.at[par]

        def abody(i, carry):
            for u in range(2):
                p = i * 2 + u
                acc = tb[p, pl.ds(0, _L)] * cb[p, pl.ds(0, _L)]
                for e in range(1, _EMBED // _L):
                    acc = acc + (tb[p, pl.ds(e * _L, _L)] *
                                 cb[p, pl.ds(e * _L, _L)])
                sbuf[p, pl.ds(0, _L)] = acc
            return carry

        lax.fori_loop(0, _CH // 2, abody, 0)

        def bbody(g, carry):
            rows = lanes + g * _L
            tot = plsc.load_gather(sbuf, [rows, jnp.full((_L,), 0,
                                                         jnp.int32)])
            for c in range(1, _L):
                tot = tot + plsc.load_gather(
                    sbuf, [rows, jnp.full((_L,), c, jnp.int32)])
            obuf[pl.ds(j * _CH + g * _L, _L)] = _log_sigmoid(tot)
            return carry

        lax.fori_loop(0, _CH // _L, bbody, 0)

    hti.wait()
    hci.wait()
    handles = [None] * _NCH
    handles[0] = fire(0, 0)
    for j in range(_NCH):
        if j + 1 < _NCH:
            handles[j + 1] = fire(j + 1, (j + 1) % 2)
        ht, hc = handles[j]
        ht.wait()
        hc.wait()
        compute(j, j % 2)

    pltpu.sync_copy(obuf, out_hbm.at[pl.ds(wid * _BPW, _BPW)])


def kernel(target_word, context_word, word_table, context_table):
    tw = target_word.reshape(_NW * _NCH, _CH)
    cw = context_word.reshape(_NW * _NCH, _CH)
    mesh = plsc.VectorSubcoreMesh(core_axis_name="c", subcore_axis_name="s",
                                  num_cores=_NC, num_subcores=_NS)
    return pl.kernel(
        _body,
        out_type=jax.ShapeDtypeStruct((_BATCH,), jnp.float32),
        mesh=mesh,
        compiler_params=pltpu.CompilerParams(needs_layout_passes=False),
        scratch_types=[
            pltpu.VMEM((_NCH, _CH), jnp.int32),
            pltpu.VMEM((_NCH, _CH), jnp.int32),
            pltpu.VMEM((2, _CH, _EMBED), jnp.float32),
            pltpu.VMEM((2, _CH, _EMBED), jnp.float32),
            pltpu.VMEM((_BPW,), jnp.float32),
            pltpu.VMEM((_CH, _L + 1), jnp.float32),
            pltpu.SemaphoreType.DMA,
            pltpu.SemaphoreType.DMA,
            pltpu.SemaphoreType.DMA,
            pltpu.SemaphoreType.DMA,
        ],
    )(tw, cw, word_table, context_table)
